# Initial kernel scaffold; baseline (speedup 1.0000x reference)
import jax
import jax.numpy as jnp
from jax.experimental import pallas as pl


def kernel(idx, spaxel_values):
    raise NotImplementedError("write your pallas kernel here")


# speedup vs baseline: 71.4536x; 71.4536x over previous
import functools

import jax
import jax.numpy as jnp
from jax import lax
from jax.experimental import pallas as pl
from jax.experimental.pallas import tpu as pltpu
from jax.experimental.pallas import tpu_sc as plsc

N_TOTAL = 4096 * 200
TABLE_SIZE = 100000
_info = plsc.get_sparse_core_info()
NC, NS, L = _info.num_cores, _info.num_subcores, _info.num_lanes
NW = NC * NS
B_PER_W = N_TOTAL // NW
VREGS_PER_W = B_PER_W // L


def _gather_body(idx_hbm, table_hbm, out_hbm, table_v, buf_v):
    wid = lax.axis_index("s") * NC + lax.axis_index("c")
    base = wid * B_PER_W

    pltpu.sync_copy(table_hbm, table_v)
    pltpu.sync_copy(idx_hbm.at[pl.ds(base, B_PER_W)], buf_v)

    def step(i, carry):
        off = i * L
        idxv = buf_v[pl.ds(off, L)]
        vals = plsc.load_gather(table_v, [idxv])
        buf_v[pl.ds(off, L)] = plsc.bitcast(vals, jnp.int32)
        return carry

    lax.fori_loop(0, VREGS_PER_W, step, 0, unroll=8)

    pltpu.sync_copy(buf_v, out_hbm.at[pl.ds(base, B_PER_W)])


@functools.partial(
    pl.kernel,
    mesh=plsc.VectorSubcoreMesh(core_axis_name="c", subcore_axis_name="s"),
    out_type=jax.ShapeDtypeStruct((N_TOTAL,), jnp.int32),
    scratch_types=[
        pltpu.VMEM((TABLE_SIZE,), jnp.float32),
        pltpu.VMEM((B_PER_W,), jnp.int32),
    ],
    compiler_params=pltpu.CompilerParams(needs_layout_passes=False),
)
def _gather_kernel(idx_hbm, table_hbm, out_hbm, table_v, buf_v):
    _gather_body(idx_hbm, table_hbm, out_hbm, table_v, buf_v)


@jax.jit
def kernel(idx, spaxel_values):
    flat_idx = idx.reshape(-1)
    out_bits = _gather_kernel(flat_idx, spaxel_values)
    return lax.bitcast_convert_type(out_bits, jnp.float32).reshape(idx.shape)


# speedup vs baseline: 85.9757x; 1.2032x over previous
import functools

import jax
import jax.numpy as jnp
from jax import lax
from jax.experimental import pallas as pl
from jax.experimental.pallas import tpu as pltpu
from jax.experimental.pallas import tpu_sc as plsc

N_ROWS = 4096
N_COLS = 200
TABLE_SIZE = 100000
_info = plsc.get_sparse_core_info()
NC, NS, L = _info.num_cores, _info.num_subcores, _info.num_lanes
NW = NC * NS
ROWS_PER_W = N_ROWS // NW
N_CHUNKS = 4
ROWS_PER_CHUNK = ROWS_PER_W // N_CHUNKS
COL_OFFSETS = tuple(range(0, N_COLS - L, L)) + (N_COLS - L,)


def _gather_body(idx_hbm, table_hbm, out_hbm, table_v, in_v, out_v):
    wid = lax.axis_index("s") * NC + lax.axis_index("c")
    row0 = wid * ROWS_PER_W

    pltpu.sync_copy(table_hbm, table_v)

    for chunk in range(N_CHUNKS):
        rb = row0 + chunk * ROWS_PER_CHUNK
        pltpu.sync_copy(idx_hbm.at[pl.ds(rb, ROWS_PER_CHUNK)], in_v)

        def row_step(r, carry):
            for off in COL_OFFSETS:
                idxv = in_v[r, pl.ds(off, L)]
                out_v[r, pl.ds(off, L)] = plsc.load_gather(table_v, [idxv])
            return carry

        lax.fori_loop(0, ROWS_PER_CHUNK, row_step, 0)

        pltpu.sync_copy(out_v, out_hbm.at[pl.ds(rb, ROWS_PER_CHUNK)])


@functools.partial(
    pl.kernel,
    mesh=plsc.VectorSubcoreMesh(core_axis_name="c", subcore_axis_name="s"),
    out_type=jax.ShapeDtypeStruct((N_ROWS, N_COLS), jnp.float32),
    scratch_types=[
        pltpu.VMEM((TABLE_SIZE,), jnp.float32),
        pltpu.VMEM((ROWS_PER_CHUNK, N_COLS), jnp.int32),
        pltpu.VMEM((ROWS_PER_CHUNK, N_COLS), jnp.float32),
    ],
    compiler_params=pltpu.CompilerParams(needs_layout_passes=False),
)
def _gather_kernel(idx_hbm, table_hbm, out_hbm, table_v, in_v, out_v):
    _gather_body(idx_hbm, table_hbm, out_hbm, table_v, in_v, out_v)


@jax.jit
def kernel(idx, spaxel_values):
    return _gather_kernel(idx, spaxel_values)


# speedup vs baseline: 101.7194x; 1.1831x over previous
import functools

import jax
import jax.numpy as jnp
from jax import lax
from jax.experimental import pallas as pl
from jax.experimental.pallas import tpu as pltpu
from jax.experimental.pallas import tpu_sc as plsc

N_ROWS = 200
N_COLS = 4096
TABLE_SIZE = 100000
_info = plsc.get_sparse_core_info()
NC, NS, L = _info.num_cores, _info.num_subcores, _info.num_lanes
NW = NC * NS
COLS_PER_W = N_COLS // NW
VREGS_PER_ROW = COLS_PER_W // L
N_CHUNKS = 5
CHUNK = N_ROWS // N_CHUNKS


def _gather_body(idx_hbm, table_hbm, out_hbm, table_v,
                 in0, in1, out0, out1, sem_t, si0, si1, so0, so1):
    wid = lax.axis_index("s") * NC + lax.axis_index("c")
    col0 = wid * COLS_PER_W

    ins, outs = (in0, in1), (out0, out1)
    sis, sos = (si0, si1), (so0, so1)

    table_cp = pltpu.async_copy(table_hbm, table_v, sem_t)

    def start_in(k):
        return pltpu.async_copy(
            idx_hbm.at[pl.ds(k * CHUNK, CHUNK), pl.ds(col0, COLS_PER_W)],
            ins[k % 2], sis[k % 2])

    def start_out(k):
        return pltpu.async_copy(
            outs[k % 2],
            out_hbm.at[pl.ds(k * CHUNK, CHUNK), pl.ds(col0, COLS_PER_W)],
            sos[k % 2])

    in_cps = {0: start_in(0), 1: start_in(1)}
    out_cps = {}
    table_cp.wait()

    for k in range(N_CHUNKS):
        in_cps[k].wait()
        if k >= 2:
            out_cps[k - 2].wait()
        iv, ov = ins[k % 2], outs[k % 2]

        def row_step(r, carry, iv=iv, ov=ov):
            for c in range(VREGS_PER_ROW):
                idxv = iv[r, pl.ds(c * L, L)]
                ov[r, pl.ds(c * L, L)] = plsc.load_gather(table_v, [idxv])
            return carry

        lax.fori_loop(0, CHUNK, row_step, 0, unroll=2)

        out_cps[k] = start_out(k)
        if k + 2 < N_CHUNKS:
            in_cps[k + 2] = start_in(k + 2)

    out_cps[N_CHUNKS - 2].wait()
    out_cps[N_CHUNKS - 1].wait()


@functools.partial(
    pl.kernel,
    mesh=plsc.VectorSubcoreMesh(core_axis_name="c", subcore_axis_name="s"),
    out_type=jax.ShapeDtypeStruct((N_ROWS, N_COLS), jnp.float32),
    scratch_types=[
        pltpu.VMEM((TABLE_SIZE,), jnp.float32),
        pltpu.VMEM((CHUNK, COLS_PER_W), jnp.int32),
        pltpu.VMEM((CHUNK, COLS_PER_W), jnp.int32),
        pltpu.VMEM((CHUNK, COLS_PER_W), jnp.float32),
        pltpu.VMEM((CHUNK, COLS_PER_W), jnp.float32),
        pltpu.SemaphoreType.DMA,
        pltpu.SemaphoreType.DMA,
        pltpu.SemaphoreType.DMA,
        pltpu.SemaphoreType.DMA,
        pltpu.SemaphoreType.DMA,
    ],
    compiler_params=pltpu.CompilerParams(needs_layout_passes=False),
)
def _gather_kernel(idx_hbm, table_hbm, out_hbm, table_v,
                   in0, in1, out0, out1, sem_t, si0, si1, so0, so1):
    _gather_body(idx_hbm, table_hbm, out_hbm, table_v,
                 in0, in1, out0, out1, sem_t, si0, si1, so0, so1)


@jax.jit
def kernel(idx, spaxel_values):
    return _gather_kernel(idx.T, spaxel_values).T


# speedup vs baseline: 124.9142x; 1.2280x over previous
import functools

import jax
import jax.numpy as jnp
from jax import lax
from jax.experimental import pallas as pl
from jax.experimental.pallas import tpu as pltpu
from jax.experimental.pallas import tpu_sc as plsc

N_ROWS = 200
N_COLS = 4096
TABLE_SIZE = 100000
_info = plsc.get_sparse_core_info()
NC, NS, L = _info.num_cores, _info.num_subcores, _info.num_lanes
NW = NC * NS
COLS_PER_W = N_COLS // NW
VREGS_PER_ROW = COLS_PER_W // L
N_CHUNKS = 5
CHUNK = N_ROWS // N_CHUNKS


def _gather_body(idx_hbm, table_hbm, out_hbm, table_v,
                 in0, in1, out0, out1, sem_t, si0, si1, so0, so1):
    wid = lax.axis_index("s") * NC + lax.axis_index("c")
    col0 = wid * COLS_PER_W

    ins, outs = (in0, in1), (out0, out1)
    sis, sos = (si0, si1), (so0, so1)

    table_cp = pltpu.async_copy(table_hbm, table_v, sem_t)

    def start_in(k):
        return pltpu.async_copy(
            idx_hbm.at[pl.ds(k * CHUNK, CHUNK), pl.ds(col0, COLS_PER_W)],
            ins[k % 2], sis[k % 2])

    def start_out(k):
        return pltpu.async_copy(
            outs[k % 2],
            out_hbm.at[pl.ds(k * CHUNK, CHUNK), pl.ds(col0, COLS_PER_W)],
            sos[k % 2])

    in_cps = {0: start_in(0), 1: start_in(1)}
    out_cps = {}
    table_cp.wait()

    for k in range(N_CHUNKS):
        in_cps[k].wait()
        if k >= 2:
            out_cps[k - 2].wait()
        iv, ov = ins[k % 2], outs[k % 2]

        @plsc.parallel_loop(0, CHUNK, unroll=4)
        def row_step(r, iv=iv, ov=ov):
            for c in range(VREGS_PER_ROW):
                idxv = iv[r, pl.ds(c * L, L)]
                ov[r, pl.ds(c * L, L)] = plsc.load_gather(table_v, [idxv])

        out_cps[k] = start_out(k)
        if k + 2 < N_CHUNKS:
            in_cps[k + 2] = start_in(k + 2)

    out_cps[N_CHUNKS - 2].wait()
    out_cps[N_CHUNKS - 1].wait()


@functools.partial(
    pl.kernel,
    mesh=plsc.VectorSubcoreMesh(core_axis_name="c", subcore_axis_name="s"),
    out_type=jax.ShapeDtypeStruct((N_ROWS, N_COLS), jnp.float32),
    scratch_types=[
        pltpu.VMEM((TABLE_SIZE,), jnp.float32),
        pltpu.VMEM((CHUNK, COLS_PER_W), jnp.int32),
        pltpu.VMEM((CHUNK, COLS_PER_W), jnp.int32),
        pltpu.VMEM((CHUNK, COLS_PER_W), jnp.float32),
        pltpu.VMEM((CHUNK, COLS_PER_W), jnp.float32),
        pltpu.SemaphoreType.DMA,
        pltpu.SemaphoreType.DMA,
        pltpu.SemaphoreType.DMA,
        pltpu.SemaphoreType.DMA,
        pltpu.SemaphoreType.DMA,
    ],
    compiler_params=pltpu.CompilerParams(needs_layout_passes=False),
)
def _gather_kernel(idx_hbm, table_hbm, out_hbm, table_v,
                   in0, in1, out0, out1, sem_t, si0, si1, so0, so1):
    _gather_body(idx_hbm, table_hbm, out_hbm, table_v,
                 in0, in1, out0, out1, sem_t, si0, si1, so0, so1)


@jax.jit
def kernel(idx, spaxel_values):
    return _gather_kernel(idx.T, spaxel_values).T


# speedup vs baseline: 125.1205x; 1.0017x over previous
import functools

import jax
import jax.numpy as jnp
from jax import lax
from jax.experimental import pallas as pl
from jax.experimental.pallas import tpu as pltpu
from jax.experimental.pallas import tpu_sc as plsc

N_ROWS = 200
N_COLS = 4096
TABLE_SIZE = 100000
_info = plsc.get_sparse_core_info()
NC, NS, L = _info.num_cores, _info.num_subcores, _info.num_lanes
NW = NC * NS
COLS_PER_W = N_COLS // NW
VREGS_PER_ROW = COLS_PER_W // L
N_CHUNKS = 5
CHUNK = N_ROWS // N_CHUNKS


def _gather_body(idx_hbm, table_hbm, out_hbm, table_v,
                 in0, in1, out0, out1, sem_t, si0, si1, so0, so1):
    wid = lax.axis_index("s") * NC + lax.axis_index("c")
    col0 = wid * COLS_PER_W

    ins, outs = (in0, in1), (out0, out1)
    sis, sos = (si0, si1), (so0, so1)

    table_cp = pltpu.async_copy(table_hbm, table_v, sem_t)

    def start_in(k):
        return pltpu.async_copy(
            idx_hbm.at[pl.ds(k * CHUNK, CHUNK), pl.ds(col0, COLS_PER_W)],
            ins[k % 2], sis[k % 2])

    def start_out(k):
        return pltpu.async_copy(
            outs[k % 2],
            out_hbm.at[pl.ds(k * CHUNK, CHUNK), pl.ds(col0, COLS_PER_W)],
            sos[k % 2])

    in_cps = {0: start_in(0), 1: start_in(1)}
    out_cps = {}
    table_cp.wait()

    for k in range(N_CHUNKS):
        in_cps[k].wait()
        if k >= 2:
            out_cps[k - 2].wait()
        iv, ov = ins[k % 2], outs[k % 2]

        @plsc.parallel_loop(0, CHUNK, unroll=4)
        def row_step(r, iv=iv, ov=ov):
            for c in range(VREGS_PER_ROW):
                idxv = iv[r, pl.ds(c * L, L)]
                ov[r, pl.ds(c * L, L)] = plsc.load_gather(table_v, [idxv])

        out_cps[k] = start_out(k)
        if k + 2 < N_CHUNKS:
            in_cps[k + 2] = start_in(k + 2)

    out_cps[N_CHUNKS - 2].wait()
    out_cps[N_CHUNKS - 1].wait()


@functools.partial(
    pl.kernel,
    mesh=plsc.VectorSubcoreMesh(core_axis_name="c", subcore_axis_name="s"),
    out_type=jax.ShapeDtypeStruct((N_ROWS, N_COLS), jnp.float32),
    scratch_types=[
        pltpu.VMEM((TABLE_SIZE,), jnp.float32),
        pltpu.VMEM((CHUNK, COLS_PER_W), jnp.int32),
        pltpu.VMEM((CHUNK, COLS_PER_W), jnp.int32),
        pltpu.VMEM((CHUNK, COLS_PER_W), jnp.float32),
        pltpu.VMEM((CHUNK, COLS_PER_W), jnp.float32),
        pltpu.SemaphoreType.DMA,
        pltpu.SemaphoreType.DMA,
        pltpu.SemaphoreType.DMA,
        pltpu.SemaphoreType.DMA,
        pltpu.SemaphoreType.DMA,
    ],
    compiler_params=pltpu.CompilerParams(
        needs_layout_passes=False, skip_device_barrier=True),
)
def _gather_kernel(idx_hbm, table_hbm, out_hbm, table_v,
                   in0, in1, out0, out1, sem_t, si0, si1, so0, so1):
    _gather_body(idx_hbm, table_hbm, out_hbm, table_v,
                 in0, in1, out0, out1, sem_t, si0, si1, so0, so1)


@jax.jit
def kernel(idx, spaxel_values):
    return _gather_kernel(idx.T, spaxel_values).T
